# baseline (speedup 1.0000x reference)
import jax
import jax.numpy as jnp
from jax import lax
from jax.experimental import pallas as pl
from jax.experimental.pallas import tpu as pltpu
from jax.experimental.pallas import tpu_sc as plsc

VOCAB = 1000000
D = 128
B = 16384
L = 20
BL = B * L

NC = 2
NS = 16
NW = NC * NS

NCH = 4
L_PER = L // NCH
CH_ROWS = BL // NCH
ROWS_PER_W = CH_ROWS // NW
CHUNK = 128
NCHUNK = ROWS_PER_W // CHUNK


def _gather_body(feat_hbm, table_hbm, emb_hbm, idx_v, rows_v, gsem):
    wid = lax.axis_index("s") * NC + lax.axis_index("c")
    base = wid * ROWS_PER_W
    pltpu.sync_copy(feat_hbm.at[wid], idx_v)

    def step(j, _):
        pltpu.async_copy(table_hbm.at[idx_v.at[j]], rows_v, gsem).wait()
        pltpu.sync_copy(rows_v, emb_hbm.at[pl.ds(base + j * CHUNK, CHUNK)])
        return _

    lax.fori_loop(0, NCHUNK, step, None)


def _sc_gather(table, feat_chunk):
    mesh = plsc.VectorSubcoreMesh(core_axis_name="c", subcore_axis_name="s")
    k = pl.kernel(
        _gather_body,
        mesh=mesh,
        out_type=jax.ShapeDtypeStruct((CH_ROWS, D), jnp.float32),
        scratch_types=[
            pltpu.VMEM((NCHUNK, CHUNK), jnp.int32),
            pltpu.VMEM((CHUNK, D), jnp.float32),
            pltpu.SemaphoreType.DMA,
        ],
    )
    return k(feat_chunk, table)


ROWS_BLK = 4096
BLKS_PER_L = B // ROWS_BLK
CH_GRID = CH_ROWS // ROWS_BLK


def _mlp_body_first(emb_ref, w1_ref, b1_ref, w2_ref, b2_ref, out_ref):
    emb = emb_ref[...].astype(jnp.bfloat16)
    h = jnp.dot(emb, w1_ref[...], preferred_element_type=jnp.float32)
    h = jnp.maximum(h + b1_ref[...], 0.0).astype(jnp.bfloat16)
    o = jnp.dot(h, w2_ref[...], preferred_element_type=jnp.float32)
    o = jnp.maximum(o + b2_ref[...], 0.0)
    out_ref[...] = o.reshape(1, ROWS_BLK, D)


def _mlp_body_chained(emb_ref, w1_ref, b1_ref, w2_ref, b2_ref, carry_ref,
                      out_ref):
    del carry_ref
    _mlp_body_first(emb_ref, w1_ref, b1_ref, w2_ref, b2_ref, out_ref)


def _tc_mlp_chunk(c, emb_c, W1, b1, W2, b2, carry):
    weight_specs = [
        pl.BlockSpec((D, 256), lambda i: (0, 0)),
        pl.BlockSpec((1, 256), lambda i: (0, 0)),
        pl.BlockSpec((256, D), lambda i: (0, 0)),
        pl.BlockSpec((1, D), lambda i: (0, 0)),
    ]
    in_specs = [pl.BlockSpec((ROWS_BLK, D), lambda i: (i, 0))] + weight_specs
    args = [emb_c, W1, b1, W2, b2]
    kwargs = {}
    body = _mlp_body_first
    if carry is not None:
        in_specs.append(pl.BlockSpec(memory_space=pl.ANY))
        args.append(carry)
        kwargs["input_output_aliases"] = {5: 0}
        body = _mlp_body_chained
    return pl.pallas_call(
        body,
        grid=(CH_GRID,),
        in_specs=in_specs,
        out_specs=pl.BlockSpec(
            (1, ROWS_BLK, D),
            lambda i, c=c: (c * L_PER + i // BLKS_PER_L, i % BLKS_PER_L, 0),
        ),
        out_shape=jax.ShapeDtypeStruct((L, B, D), jnp.float32),
        **kwargs,
    )(*args)


def kernel(features, table, W1, b1, W2, b2):
    feat = features.T.reshape(-1).astype(jnp.int32)
    feat = feat.reshape(NCH, NW, NCHUNK, CHUNK)
    b1r = b1.reshape(1, 256)
    b2r = b2.reshape(1, 128)
    W1b = W1.astype(jnp.bfloat16)
    W2b = W2.astype(jnp.bfloat16)
    embs = [_sc_gather(table, feat[c]) for c in range(NCH)]
    out = None
    for c in range(NCH):
        out = _tc_mlp_chunk(c, embs[c], W1b, b1r, W2b, b2r, out)
    return jnp.transpose(out, (1, 0, 2))


# speedup vs baseline: 1.1344x; 1.1344x over previous
import numpy as np

import jax
import jax.numpy as jnp
from jax import lax
from jax.experimental import pallas as pl
from jax.experimental.pallas import tpu as pltpu
from jax.experimental.pallas import tpu_sc as plsc

VOCAB = 1000000
D = 128
B = 16384
L = 20
BL = B * L

NC = 2
NS = 16
NW = NC * NS

NCH = 4
L_PER = L // NCH
CH_ROWS = BL // NCH
ROWS_PER_W = CH_ROWS // NW
CHUNK = 128
NCHUNK = ROWS_PER_W // CHUNK
NPAIR = NCHUNK // 2
PAIRS_PER_W = ROWS_PER_W // 2
REGION = 8192
HALF = REGION // 2


def _rne_bf16_bits(vref, r, g):
    u = lax.bitcast_convert_type(vref[r, pl.ds(16 * g, 16)], jnp.uint32)
    return (u + jnp.uint32(0x7FFF) + ((u >> 16) & jnp.uint32(1))) >> 16


def _gather_body(feat_hbm, table_hbm, emb_hbm, idx_v, lo_v, hi_v, u_v,
                 sem0, sem1):
    wid = lax.axis_index("s") * NC + lax.axis_index("c")
    base = wid * PAIRS_PER_W
    pltpu.sync_copy(feat_hbm.at[wid], idx_v)
    sems = (sem0, sem1)

    def start_pair(t, par):
        pltpu.make_async_copy(
            table_hbm.at[idx_v.at[2 * t]], lo_v.at[par], sems[par]).start()
        pltpu.make_async_copy(
            table_hbm.at[idx_v.at[2 * t + 1]], hi_v.at[par], sems[par]).start()

    def wait_pair(t, par):
        pltpu.make_async_copy(
            table_hbm.at[idx_v.at[2 * t]], lo_v.at[par], sems[par]).wait()
        pltpu.make_async_copy(
            table_hbm.at[idx_v.at[2 * t + 1]], hi_v.at[par], sems[par]).wait()

    start_pair(0, 0)

    def step(tt, _):
        for par in (0, 1):
            t = 2 * tt + par
            wait_pair(t, par)
            start_pair(jnp.minimum(t + 1, NPAIR - 1), 1 - par)

            def conv_row(r, _c):
                for g in range(D // 16):
                    a = _rne_bf16_bits(lo_v.at[par], r, g)
                    b = _rne_bf16_bits(hi_v.at[par], r, g)
                    u_v[r, pl.ds(16 * g, 16)] = lax.bitcast_convert_type(
                        a | (b << 16), jnp.int32)
                return _c

            lax.fori_loop(0, CHUNK, conv_row, 0)
            pltpu.sync_copy(u_v, emb_hbm.at[pl.ds(base + t * CHUNK, CHUNK)])
        return _

    lax.fori_loop(0, NPAIR // 2, step, None)
    wait_pair(NPAIR - 1, 0)


def _sc_gather(table, feat_chunk):
    mesh = plsc.VectorSubcoreMesh(core_axis_name="c", subcore_axis_name="s")
    k = pl.kernel(
        _gather_body,
        mesh=mesh,
        out_type=jax.ShapeDtypeStruct((CH_ROWS // 2, D), jnp.int32),
        scratch_types=[
            pltpu.VMEM((NCHUNK, CHUNK), jnp.int32),
            pltpu.VMEM((2, CHUNK, D), jnp.float32),
            pltpu.VMEM((2, CHUNK, D), jnp.float32),
            pltpu.VMEM((CHUNK, D), jnp.int32),
            pltpu.SemaphoreType.DMA,
            pltpu.SemaphoreType.DMA,
        ],
    )
    return k(feat_chunk, table)


CH_GRID = CH_ROWS // REGION
B_BLKS = B // REGION


def _half_mlp(x_f32, w1_ref, b1_ref, w2_ref, b2_ref):
    x = x_f32.astype(jnp.bfloat16)
    h = jnp.dot(x, w1_ref[...], preferred_element_type=jnp.float32)
    h = jnp.maximum(h + b1_ref[...], 0.0).astype(jnp.bfloat16)
    o = jnp.dot(h, w2_ref[...], preferred_element_type=jnp.float32)
    return jnp.maximum(o + b2_ref[...], 0.0)


def _mlp_body_first(emb_ref, w1_ref, b1_ref, w2_ref, b2_ref, out_ref):
    w = emb_ref[...]
    lo = lax.bitcast_convert_type(w << 16, jnp.float32)
    hi = lax.bitcast_convert_type(
        w & jnp.int32(np.int32(np.uint32(0xFFFF0000).view(np.int32))),
        jnp.float32)
    o_lo = _half_mlp(lo, w1_ref, b1_ref, w2_ref, b2_ref)
    o_hi = _half_mlp(hi, w1_ref, b1_ref, w2_ref, b2_ref)
    out_ref[0, 0:HALF, :] = o_lo
    out_ref[0, HALF:REGION, :] = o_hi


def _mlp_body_chained(emb_ref, w1_ref, b1_ref, w2_ref, b2_ref, carry_ref,
                      out_ref):
    del carry_ref
    _mlp_body_first(emb_ref, w1_ref, b1_ref, w2_ref, b2_ref, out_ref)


def _tc_mlp_chunk(c, emb_c, W1, b1, W2, b2, carry):
    weight_specs = [
        pl.BlockSpec((D, 256), lambda i: (0, 0)),
        pl.BlockSpec((1, 256), lambda i: (0, 0)),
        pl.BlockSpec((256, D), lambda i: (0, 0)),
        pl.BlockSpec((1, D), lambda i: (0, 0)),
    ]
    in_specs = [pl.BlockSpec((HALF, D), lambda i: (i, 0))] + weight_specs
    args = [emb_c, W1, b1, W2, b2]
    kwargs = {}
    body = _mlp_body_first
    if carry is not None:
        in_specs.append(pl.BlockSpec(memory_space=pl.ANY))
        args.append(carry)
        kwargs["input_output_aliases"] = {5: 0}
        body = _mlp_body_chained
    return pl.pallas_call(
        body,
        grid=(CH_GRID,),
        in_specs=in_specs,
        out_specs=pl.BlockSpec(
            (1, REGION, D),
            lambda i, c=c: (c * L_PER + i // B_BLKS, i % B_BLKS, 0),
        ),
        out_shape=jax.ShapeDtypeStruct((L, B, D), jnp.float32),
        **kwargs,
    )(*args)


def _pair_order() -> np.ndarray:
    order = np.empty((NCH, NW, NPAIR, 2, CHUNK), np.int32)
    j = np.arange(CHUNK, dtype=np.int32)
    for c in range(NCH):
        for w in range(NW):
            for t in range(NPAIR):
                q0 = PAIRS_PER_W * w + CHUNK * t
                blk8, i = divmod(q0, HALF)
                lo = c * CH_ROWS + blk8 * REGION + i
                order[c, w, t, 0] = lo + j
                order[c, w, t, 1] = lo + HALF + j
    return order.reshape(-1)


_ORDER = _pair_order()


def kernel(features, table, W1, b1, W2, b2):
    feat = features.T.reshape(-1).astype(jnp.int32)
    feat = feat[_ORDER].reshape(NCH, NW, NCHUNK, CHUNK)
    b1r = b1.reshape(1, 256)
    b2r = b2.reshape(1, 128)
    W1b = W1.astype(jnp.bfloat16)
    W2b = W2.astype(jnp.bfloat16)
    embs = [_sc_gather(table, feat[c]) for c in range(NCH)]
    out = None
    for c in range(NCH):
        out = _tc_mlp_chunk(c, embs[c], W1b, b1r, W2b, b2r, out)
    return jnp.transpose(out, (1, 0, 2))


# speedup vs baseline: 1.2831x; 1.1311x over previous
import numpy as np

import jax
import jax.numpy as jnp
from jax import lax
from jax.experimental import pallas as pl
from jax.experimental.pallas import tpu as pltpu
from jax.experimental.pallas import tpu_sc as plsc

VOCAB = 1000000
D = 128
B = 16384
L = 20
BL = B * L

NC = 2
NS = 16
NW = NC * NS

NCH = 5
L_PER = L // NCH
CH_ROWS = BL // NCH
ROWS_PER_W = CH_ROWS // NW
CHUNK = 128
NCHUNK = ROWS_PER_W // CHUNK
NPAIR = NCHUNK // 2
PAIRS_PER_W = ROWS_PER_W // 2
REGION = 8192
HALF = REGION // 2


def _rne_bf16_bits(vref, r, g):
    u = lax.bitcast_convert_type(vref[r, pl.ds(16 * g, 16)], jnp.uint32)
    return (u + jnp.uint32(0x7FFF) + ((u >> 16) & jnp.uint32(1))) >> 16


def _gather_body(feat_hbm, table_hbm, emb_hbm, idx_v, lo_v, hi_v, u_v,
                 sem0, sem1):
    wid = lax.axis_index("s") * NC + lax.axis_index("c")
    base = wid * PAIRS_PER_W
    pltpu.sync_copy(feat_hbm.at[wid], idx_v)
    sems = (sem0, sem1)

    def start_pair(t, par):
        pltpu.make_async_copy(
            table_hbm.at[idx_v.at[2 * t]], lo_v.at[par], sems[par]).start()
        pltpu.make_async_copy(
            table_hbm.at[idx_v.at[2 * t + 1]], hi_v.at[par], sems[par]).start()

    def wait_pair(t, par):
        pltpu.make_async_copy(
            table_hbm.at[idx_v.at[2 * t]], lo_v.at[par], sems[par]).wait()
        pltpu.make_async_copy(
            table_hbm.at[idx_v.at[2 * t + 1]], hi_v.at[par], sems[par]).wait()

    start_pair(0, 0)

    def step(tt, _):
        for par in (0, 1):
            t = 2 * tt + par
            wait_pair(t, par)
            start_pair(jnp.minimum(t + 1, NPAIR - 1), 1 - par)

            def conv_row(r, _c):
                for g in range(D // 16):
                    a = _rne_bf16_bits(lo_v.at[par], r, g)
                    b = _rne_bf16_bits(hi_v.at[par], r, g)
                    u_v[r, pl.ds(16 * g, 16)] = lax.bitcast_convert_type(
                        a | (b << 16), jnp.int32)
                return _c

            lax.fori_loop(0, CHUNK, conv_row, 0)
            pltpu.sync_copy(u_v, emb_hbm.at[pl.ds(base + t * CHUNK, CHUNK)])
        return _

    lax.fori_loop(0, NPAIR // 2, step, None)
    wait_pair(NPAIR - 1, 0)


def _sc_gather(table, feat_chunk):
    mesh = plsc.VectorSubcoreMesh(core_axis_name="c", subcore_axis_name="s")
    k = pl.kernel(
        _gather_body,
        mesh=mesh,
        out_type=jax.ShapeDtypeStruct((CH_ROWS // 2, D), jnp.int32),
        scratch_types=[
            pltpu.VMEM((NCHUNK, CHUNK), jnp.int32),
            pltpu.VMEM((2, CHUNK, D), jnp.float32),
            pltpu.VMEM((2, CHUNK, D), jnp.float32),
            pltpu.VMEM((CHUNK, D), jnp.int32),
            pltpu.SemaphoreType.DMA,
            pltpu.SemaphoreType.DMA,
        ],
    )
    return k(feat_chunk, table)


CH_GRID = CH_ROWS // REGION
NB8 = CH_ROWS // REGION
B_BLKS = B // REGION


def _half_mlp(x_f32, w1_ref, b1_ref, w2_ref, b2_ref):
    x = x_f32.astype(jnp.bfloat16)
    h = jnp.dot(x, w1_ref[...], preferred_element_type=jnp.float32)
    h = jnp.maximum(h + b1_ref[...], 0.0).astype(jnp.bfloat16)
    o = jnp.dot(h, w2_ref[...], preferred_element_type=jnp.float32)
    return jnp.maximum(o + b2_ref[...], 0.0)


def _mlp_body_first(emb_ref, w1_ref, b1_ref, w2_ref, b2_ref, out_ref):
    w = emb_ref[...]
    lo = lax.bitcast_convert_type(w << 16, jnp.float32)
    hi = lax.bitcast_convert_type(
        w & jnp.int32(np.int32(np.uint32(0xFFFF0000).view(np.int32))),
        jnp.float32)
    o_lo = _half_mlp(lo, w1_ref, b1_ref, w2_ref, b2_ref)
    o_hi = _half_mlp(hi, w1_ref, b1_ref, w2_ref, b2_ref)
    out_ref[0, 0:HALF, :] = o_lo
    out_ref[0, HALF:REGION, :] = o_hi


def _mlp_body_chained(emb_ref, w1_ref, b1_ref, w2_ref, b2_ref, carry_ref,
                      out_ref):
    del carry_ref
    _mlp_body_first(emb_ref, w1_ref, b1_ref, w2_ref, b2_ref, out_ref)


def _tc_mlp_chunk(c, emb_c, W1, b1, W2, b2, carry):
    weight_specs = [
        pl.BlockSpec((D, 256), lambda i: (0, 0)),
        pl.BlockSpec((1, 256), lambda i: (0, 0)),
        pl.BlockSpec((256, D), lambda i: (0, 0)),
        pl.BlockSpec((1, D), lambda i: (0, 0)),
    ]
    in_specs = [pl.BlockSpec((HALF, D), lambda i: (i, 0))] + weight_specs
    args = [emb_c, W1, b1, W2, b2]
    kwargs = {}
    body = _mlp_body_first
    if carry is not None:
        in_specs.append(pl.BlockSpec(memory_space=pl.ANY))
        args.append(carry)
        kwargs["input_output_aliases"] = {5: 0}
        body = _mlp_body_chained
    return pl.pallas_call(
        body,
        grid=(CH_GRID,),
        in_specs=in_specs,
        out_specs=pl.BlockSpec(
            (1, REGION, D),
            lambda i, c=c: (c * L_PER + i // B_BLKS, i % B_BLKS, 0),
        ),
        out_shape=jax.ShapeDtypeStruct((L, B, D), jnp.float32),
        **kwargs,
    )(*args)


def kernel(features, table, W1, b1, W2, b2):
    feat = features.T.reshape(-1).astype(jnp.int32)
    feat = (feat.reshape(NCH, NB8, 2, HALF // CHUNK, CHUNK)
                .transpose(0, 1, 3, 2, 4)
                .reshape(NCH, NW, NCHUNK, CHUNK))
    b1r = b1.reshape(1, 256)
    b2r = b2.reshape(1, 128)
    W1b = W1.astype(jnp.bfloat16)
    W2b = W2.astype(jnp.bfloat16)
    embs = [_sc_gather(table, feat[c]) for c in range(NCH)]
    out = None
    for c in range(NCH):
        out = _tc_mlp_chunk(c, embs[c], W1b, b1r, W2b, b2r, out)
    return jnp.transpose(out, (1, 0, 2))
